# baseline (speedup 1.0000x reference)
import functools

import jax
import jax.numpy as jnp
from jax import lax
from jax.experimental import pallas as pl
from jax.experimental.pallas import tpu as pltpu
from jax.experimental.pallas import tpu_sc as plsc

_B = 4096
_T = 50
_D = 32
_ATT_H = 80
_LANES = 128
_NW = 32
_HROWS = _T * _B
_IDXROWS = _HROWS // _LANES
_WROWS = _IDXROWS // _NW
_CHUNK = 10
_NCHUNK = _WROWS // _CHUNK
_BB = 1024


def _sc_gather(idx_cat, idx_brand, idx_qc, idx_qb, idx_u, idx_i,
               cat_t, brand_t, user_t, item_t):
  f32 = jnp.float32
  mesh = plsc.VectorSubcoreMesh(core_axis_name="c", subcore_axis_name="s")
  out_type = [
      jax.ShapeDtypeStruct((_HROWS, _D), f32),
      jax.ShapeDtypeStruct((_HROWS, _D), f32),
      jax.ShapeDtypeStruct((_B, _D), f32),
      jax.ShapeDtypeStruct((_B, _D), f32),
      jax.ShapeDtypeStruct((_B, _D), f32),
      jax.ShapeDtypeStruct((_B, _D), f32),
  ]

  @functools.partial(
      pl.kernel, mesh=mesh, out_type=out_type,
      compiler_params=pltpu.CompilerParams(use_tc_tiling_on_sc=False),
      scratch_types=[
          pltpu.VMEM((_WROWS, _LANES), jnp.int32),
          pltpu.VMEM((_CHUNK * _LANES, _D), f32),
          pltpu.VMEM((1, _LANES), jnp.int32),
          pltpu.VMEM((_LANES, _D), f32),
          pltpu.SemaphoreType.DMA,
      ])
  def body(idx_cat_h, idx_brand_h, idx_qc_h, idx_qb_h, idx_u_h, idx_i_h,
           cat_h, brand_h, user_h, item_h,
           kcat_h, kbrand_h, qc_h, qb_h, u_h, i_h,
           idx_v, rows_v, idx_s, rows_s, sem):
    wid = lax.axis_index("s") * 2 + lax.axis_index("c")
    w0 = wid * _WROWS

    def run_table(idx_h, tab_h, out_h):
      pltpu.sync_copy(idx_h.at[wid], idx_v)

      def chunk(c, carry):
        handles = [
            pltpu.async_copy(tab_h.at[idx_v.at[c * _CHUNK + j]],
                             rows_v.at[pl.ds(j * _LANES, _LANES)], sem)
            for j in range(_CHUNK)
        ]
        for h in handles:
          h.wait()
        pltpu.sync_copy(
            rows_v,
            out_h.at[pl.ds((w0 + c * _CHUNK) * _LANES, _CHUNK * _LANES)])
        return carry
      lax.fori_loop(0, _NCHUNK, chunk, 0)

    run_table(idx_cat_h, cat_h, kcat_h)
    run_table(idx_brand_h, brand_h, kbrand_h)

    for idx_h, tab_h, out_h in ((idx_qc_h, cat_h, qc_h),
                                (idx_qb_h, brand_h, qb_h),
                                (idx_u_h, user_h, u_h),
                                (idx_i_h, item_h, i_h)):
      pltpu.sync_copy(idx_h.at[wid], idx_s)
      pltpu.async_copy(tab_h.at[idx_s.at[0]], rows_s, sem).wait()
      pltpu.sync_copy(rows_s, out_h.at[pl.ds(wid * _LANES, _LANES)])

  return body(idx_cat, idx_brand, idx_qc, idx_qb, idx_u, idx_i,
              cat_t, brand_t, user_t, item_t)


def _tc_body(kcat_ref, kbrand_ref, qc_ref, qb_ref, u_ref, i_ref, dense_ref,
             wa_ref, wcomb_ref, b1_ref, w2_ref, b2_ref,
             w1a_ref, w1b_ref, w1c_ref, w1d_ref, w1e_ref, db1_ref,
             w2d_ref, db2_ref, w3_ref, db3_ref, out_ref):
  f32 = jnp.float32
  dot = functools.partial(jnp.dot, preferred_element_type=f32)
  qT = jnp.concatenate([qc_ref[...], qb_ref[...]], axis=0)
  qA = dot(wa_ref[...], qT) + b1_ref[...]
  w2 = w2_ref[...]
  b2 = b2_ref[0, 0]

  m = jnp.full((1, _BB), -1e30, f32)
  den = jnp.zeros((1, _BB), f32)
  acc = jnp.zeros((2 * _D, _BB), f32)
  for t in range(_T):
    ktT = jnp.concatenate([kcat_ref[t], kbrand_ref[t]], axis=0)
    kk = jnp.concatenate([ktT, qT * ktT], axis=0)
    h = jnp.maximum(qA + dot(wcomb_ref[...], kk), 0.0)
    s = jnp.sum(h * w2, axis=0, keepdims=True) + b2
    m2 = jnp.maximum(m, s)
    alpha = jnp.exp(m - m2)
    p = jnp.exp(s - m2)
    den = den * alpha + p
    acc = acc * alpha + p * ktT
    m = m2
  att = acc / den

  h1 = jnp.maximum(
      dot(w1a_ref[...], dense_ref[...]) + dot(w1b_ref[...], qT)
      + dot(w1c_ref[...], att) + dot(w1d_ref[...], u_ref[...])
      + dot(w1e_ref[...], i_ref[...]) + db1_ref[...], 0.0)
  h2 = jnp.maximum(dot(w2d_ref[...], h1) + db2_ref[...], 0.0)
  z = jnp.sum(h2 * w3_ref[...], axis=0, keepdims=True) + db3_ref[0, 0]
  out_ref[...] = 1.0 / (1.0 + jnp.exp(-z))


def _full(shape):
  return pl.BlockSpec(shape, lambda i: (0,) * len(shape))


def kernel(dense_features, item_category, item_brand, user_pv_category_list,
           user_pv_brand_list, user_id, item_id,
           category_table, brand_table, user_table, item_table,
           att_w1, att_b1, att_w2, att_b2,
           dnn_w1, dnn_b1, dnn_w2, dnn_b2, dnn_w3, dnn_b3):
  i32 = jnp.int32
  idx_cat = user_pv_category_list.astype(i32).T.reshape(_NW, _WROWS, _LANES)
  idx_brand = user_pv_brand_list.astype(i32).T.reshape(_NW, _WROWS, _LANES)
  idx_qc = item_category.astype(i32).reshape(_NW, 1, _LANES)
  idx_qb = item_brand.astype(i32).reshape(_NW, 1, _LANES)
  idx_u = user_id.astype(i32).reshape(_NW, 1, _LANES)
  idx_i = item_id.astype(i32).reshape(_NW, 1, _LANES)

  kcat, kbrand, qc, qb, u_emb, i_emb = _sc_gather(
      idx_cat, idx_brand, idx_qc, idx_qb, idx_u, idx_i,
      category_table, brand_table, user_table, item_table)

  kcatT = kcat.reshape(_T, _B, _D).transpose(0, 2, 1)
  kbrandT = kbrand.reshape(_T, _B, _D).transpose(0, 2, 1)
  qcT = qc.T
  qbT = qb.T
  uT = u_emb.T
  iT = i_emb.T
  denseT = dense_features.T

  wd = att_w1[4 * _D:6 * _D]
  wa_t = (att_w1[0:2 * _D] + wd).T
  wcomb_t = jnp.concatenate(
      [(att_w1[2 * _D:4 * _D] - wd).T, att_w1[6 * _D:8 * _D].T],
      axis=1)
  b1c = att_b1.reshape(_ATT_H, 1)
  w2c = att_w2.reshape(_ATT_H, 1)
  b2c = att_b2.reshape(1, 1)
  w1a = dnn_w1[0:13].T
  w1b = dnn_w1[13:13 + 64].T
  w1c = dnn_w1[77:77 + 64].T
  w1d = dnn_w1[141:141 + 32].T
  w1e = dnn_w1[173:173 + 32].T
  db1c = dnn_b1.reshape(-1, 1)
  w2d = dnn_w2.T
  db2c = dnn_b2.reshape(-1, 1)
  w3c = dnn_w3.reshape(-1, 1)
  db3c = dnn_b3.reshape(1, 1)

  grid = (_B // _BB,)
  outT = pl.pallas_call(
      _tc_body,
      grid=grid,
      in_specs=[
          pl.BlockSpec((_T, _D, _BB), lambda i: (0, 0, i)),
          pl.BlockSpec((_T, _D, _BB), lambda i: (0, 0, i)),
          pl.BlockSpec((_D, _BB), lambda i: (0, i)),
          pl.BlockSpec((_D, _BB), lambda i: (0, i)),
          pl.BlockSpec((_D, _BB), lambda i: (0, i)),
          pl.BlockSpec((_D, _BB), lambda i: (0, i)),
          pl.BlockSpec((13, _BB), lambda i: (0, i)),
          _full(wa_t.shape),
          _full(wcomb_t.shape),
          _full(b1c.shape),
          _full(w2c.shape),
          _full(b2c.shape),
          _full(w1a.shape),
          _full(w1b.shape),
          _full(w1c.shape),
          _full(w1d.shape),
          _full(w1e.shape),
          _full(db1c.shape),
          _full(w2d.shape),
          _full(db2c.shape),
          _full(w3c.shape),
          _full(db3c.shape),
      ],
      out_specs=pl.BlockSpec((1, _BB), lambda i: (0, i)),
      out_shape=jax.ShapeDtypeStruct((1, _B), jnp.float32),
  )(kcatT, kbrandT, qcT, qbT, uT, iT, denseT,
    wa_t, wcomb_t, b1c, w2c, b2c,
    w1a, w1b, w1c, w1d, w1e, db1c, w2d, db2c, w3c, db3c)
  return outT.reshape(_B, 1)


# speedup vs baseline: 1.0698x; 1.0698x over previous
import functools

import jax
import jax.numpy as jnp
from jax import lax
from jax.experimental import pallas as pl
from jax.experimental.pallas import tpu as pltpu
from jax.experimental.pallas import tpu_sc as plsc

_B = 4096
_T = 50
_D = 32
_ATT_H = 80
_LANES = 128
_NW = 32
_HROWS = _T * _B
_IDXROWS = _HROWS // _LANES
_WROWS = _IDXROWS // _NW
_CHUNK = 10
_NCHUNK = _WROWS // _CHUNK
_BB = 512


def _sc_gather_hist(idx_cat, idx_brand, cat_t, brand_t):
  f32 = jnp.float32
  mesh = plsc.VectorSubcoreMesh(core_axis_name="c", subcore_axis_name="s")
  out_type = [
      jax.ShapeDtypeStruct((_HROWS, _D), f32),
      jax.ShapeDtypeStruct((_HROWS, _D), f32),
  ]

  @functools.partial(
      pl.kernel, mesh=mesh, out_type=out_type,
      compiler_params=pltpu.CompilerParams(use_tc_tiling_on_sc=False),
      scratch_types=[
          pltpu.VMEM((_WROWS, _LANES), jnp.int32),
          pltpu.VMEM((_CHUNK * _LANES, _D), f32),
          pltpu.SemaphoreType.DMA,
      ])
  def body(idx_cat_h, idx_brand_h, cat_h, brand_h,
           kcat_h, kbrand_h, idx_v, rows_v, sem):
    wid = lax.axis_index("s") * 2 + lax.axis_index("c")
    w0 = wid * _WROWS

    def run_table(idx_h, tab_h, out_h):
      pltpu.sync_copy(idx_h.at[wid], idx_v)

      def chunk(c, carry):
        handles = [
            pltpu.async_copy(tab_h.at[idx_v.at[c * _CHUNK + j]],
                             rows_v.at[pl.ds(j * _LANES, _LANES)], sem)
            for j in range(_CHUNK)
        ]
        for h in handles:
          h.wait()
        pltpu.sync_copy(
            rows_v,
            out_h.at[pl.ds((w0 + c * _CHUNK) * _LANES, _CHUNK * _LANES)])
        return carry
      lax.fori_loop(0, _NCHUNK, chunk, 0)

    run_table(idx_cat_h, cat_h, kcat_h)
    run_table(idx_brand_h, brand_h, kbrand_h)

  return body(idx_cat, idx_brand, cat_t, brand_t)


def _sc_gather_small(idx_qc, idx_qb, idx_u, idx_i,
                     cat_t, brand_t, user_t, item_t):
  f32 = jnp.float32
  mesh = plsc.VectorSubcoreMesh(core_axis_name="c", subcore_axis_name="s")
  out_type = [jax.ShapeDtypeStruct((_B, _D), f32)] * 4

  @functools.partial(
      pl.kernel, mesh=mesh, out_type=out_type,
      compiler_params=pltpu.CompilerParams(use_tc_tiling_on_sc=False),
      scratch_types=[
          pltpu.VMEM((1, _LANES), jnp.int32),
          pltpu.VMEM((_LANES, _D), f32),
          pltpu.SemaphoreType.DMA,
      ])
  def body(idx_qc_h, idx_qb_h, idx_u_h, idx_i_h,
           cat_h, brand_h, user_h, item_h,
           qc_h, qb_h, u_h, i_h, idx_s, rows_s, sem):
    wid = lax.axis_index("s") * 2 + lax.axis_index("c")
    for idx_h, tab_h, out_h in ((idx_qc_h, cat_h, qc_h),
                                (idx_qb_h, brand_h, qb_h),
                                (idx_u_h, user_h, u_h),
                                (idx_i_h, item_h, i_h)):
      pltpu.sync_copy(idx_h.at[wid], idx_s)
      pltpu.async_copy(tab_h.at[idx_s.at[0]], rows_s, sem).wait()
      pltpu.sync_copy(rows_s, out_h.at[pl.ds(wid * _LANES, _LANES)])

  return body(idx_qc, idx_qb, idx_u, idx_i, cat_t, brand_t, user_t, item_t)


def _tc_body(kcat_ref, kbrand_ref, qc_ref, qb_ref, u_ref, i_ref, dense_ref,
             wa_ref, wcomb_ref, b1_ref, w2_ref, b2_ref,
             w1a_ref, w1b_ref, w1c_ref, w1d_ref, w1e_ref, db1_ref,
             w2d_ref, db2_ref, w3_ref, db3_ref, out_ref):
  f32 = jnp.float32
  dot = functools.partial(jnp.dot, preferred_element_type=f32)
  qT = jnp.concatenate([qc_ref[...], qb_ref[...]], axis=0)
  qA = dot(wa_ref[...], qT) + b1_ref[...]
  w2 = w2_ref[...]
  b2 = b2_ref[0, 0]

  m = jnp.full((1, _BB), -1e30, f32)
  den = jnp.zeros((1, _BB), f32)
  acc = jnp.zeros((2 * _D, _BB), f32)
  for t in range(_T):
    ktT = jnp.concatenate([kcat_ref[t], kbrand_ref[t]], axis=0)
    kk = jnp.concatenate([ktT, qT * ktT], axis=0)
    h = jnp.maximum(qA + dot(wcomb_ref[...], kk), 0.0)
    s = jnp.sum(h * w2, axis=0, keepdims=True) + b2
    m2 = jnp.maximum(m, s)
    alpha = jnp.exp(m - m2)
    p = jnp.exp(s - m2)
    den = den * alpha + p
    acc = acc * alpha + p * ktT
    m = m2
  att = acc / den

  h1 = jnp.maximum(
      dot(w1a_ref[...], dense_ref[...]) + dot(w1b_ref[...], qT)
      + dot(w1c_ref[...], att) + dot(w1d_ref[...], u_ref[...])
      + dot(w1e_ref[...], i_ref[...]) + db1_ref[...], 0.0)
  h2 = jnp.maximum(dot(w2d_ref[...], h1) + db2_ref[...], 0.0)
  z = jnp.sum(h2 * w3_ref[...], axis=0, keepdims=True) + db3_ref[0, 0]
  out_ref[...] = 1.0 / (1.0 + jnp.exp(-z))


def _full(shape):
  return pl.BlockSpec(shape, lambda i: (0,) * len(shape))


def kernel(dense_features, item_category, item_brand, user_pv_category_list,
           user_pv_brand_list, user_id, item_id,
           category_table, brand_table, user_table, item_table,
           att_w1, att_b1, att_w2, att_b2,
           dnn_w1, dnn_b1, dnn_w2, dnn_b2, dnn_w3, dnn_b3):
  i32 = jnp.int32
  idx_cat = user_pv_category_list.astype(i32).T.reshape(_NW, _WROWS, _LANES)
  idx_brand = user_pv_brand_list.astype(i32).T.reshape(_NW, _WROWS, _LANES)
  idx_qc = item_category.astype(i32).reshape(_NW, 1, _LANES)
  idx_qb = item_brand.astype(i32).reshape(_NW, 1, _LANES)
  idx_u = user_id.astype(i32).reshape(_NW, 1, _LANES)
  idx_i = item_id.astype(i32).reshape(_NW, 1, _LANES)

  kcat, kbrand = _sc_gather_hist(idx_cat, idx_brand,
                                 category_table, brand_table)
  qc, qb, u_emb, i_emb = _sc_gather_small(
      idx_qc, idx_qb, idx_u, idx_i,
      category_table, brand_table, user_table, item_table)

  kcatT = kcat.reshape(_T, _B, _D).transpose(0, 2, 1)
  kbrandT = kbrand.reshape(_T, _B, _D).transpose(0, 2, 1)
  qcT = qc.T
  qbT = qb.T
  uT = u_emb.T
  iT = i_emb.T
  denseT = dense_features.T

  wd = att_w1[4 * _D:6 * _D]
  wa_t = (att_w1[0:2 * _D] + wd).T
  wcomb_t = jnp.concatenate(
      [(att_w1[2 * _D:4 * _D] - wd).T, att_w1[6 * _D:8 * _D].T],
      axis=1)
  b1c = att_b1.reshape(_ATT_H, 1)
  w2c = att_w2.reshape(_ATT_H, 1)
  b2c = att_b2.reshape(1, 1)
  w1a = dnn_w1[0:13].T
  w1b = dnn_w1[13:13 + 64].T
  w1c = dnn_w1[77:77 + 64].T
  w1d = dnn_w1[141:141 + 32].T
  w1e = dnn_w1[173:173 + 32].T
  db1c = dnn_b1.reshape(-1, 1)
  w2d = dnn_w2.T
  db2c = dnn_b2.reshape(-1, 1)
  w3c = dnn_w3.reshape(-1, 1)
  db3c = dnn_b3.reshape(1, 1)

  grid = (_B // _BB,)
  outT = pl.pallas_call(
      _tc_body,
      grid=grid,
      in_specs=[
          pl.BlockSpec((_T, _D, _BB), lambda i: (0, 0, i)),
          pl.BlockSpec((_T, _D, _BB), lambda i: (0, 0, i)),
          pl.BlockSpec((_D, _BB), lambda i: (0, i)),
          pl.BlockSpec((_D, _BB), lambda i: (0, i)),
          pl.BlockSpec((_D, _BB), lambda i: (0, i)),
          pl.BlockSpec((_D, _BB), lambda i: (0, i)),
          pl.BlockSpec((13, _BB), lambda i: (0, i)),
          _full(wa_t.shape),
          _full(wcomb_t.shape),
          _full(b1c.shape),
          _full(w2c.shape),
          _full(b2c.shape),
          _full(w1a.shape),
          _full(w1b.shape),
          _full(w1c.shape),
          _full(w1d.shape),
          _full(w1e.shape),
          _full(db1c.shape),
          _full(w2d.shape),
          _full(db2c.shape),
          _full(w3c.shape),
          _full(db3c.shape),
      ],
      out_specs=pl.BlockSpec((1, _BB), lambda i: (0, i)),
      out_shape=jax.ShapeDtypeStruct((1, _B), jnp.float32),
  )(kcatT, kbrandT, qcT, qbT, uT, iT, denseT,
    wa_t, wcomb_t, b1c, w2c, b2c,
    w1a, w1b, w1c, w1d, w1e, db1c, w2d, db2c, w3c, db3c)
  return outT.reshape(_B, 1)
